# Initial kernel scaffold; baseline (speedup 1.0000x reference)
import jax
import jax.numpy as jnp
from jax.experimental import pallas as pl


def kernel(x, regression, classification, anchors):
    raise NotImplementedError("write your pallas kernel here")


# speedup vs baseline: 41.4005x; 41.4005x over previous
import jax
import jax.numpy as jnp
from jax import lax
from jax.experimental import pallas as pl
from jax.experimental.pallas import tpu as pltpu

_N = 5000
_NP = 5120
_B = 512
_NB = _NP // _B
_NC = 90
_CP = 128
_ST = 0.05
_TH = 0.5


def _iota(shape, dim):
    return lax.broadcasted_iota(jnp.int32, shape, dim)


def _fiota(shape, dim):
    return _iota(shape, dim).astype(jnp.float32)


def _boxes_from(ax1, ay1, ax2, ay2, dx, dy, dw, dh, img_w, img_h):
    w = ax2 - ax1
    h = ay2 - ay1
    cx = ax1 + 0.5 * w
    cy = ay1 + 0.5 * h
    pcx = cx + dx * 0.1 * w
    pcy = cy + dy * 0.1 * h
    pw = jnp.exp(dw * 0.2) * w
    ph = jnp.exp(dh * 0.2) * h
    x1 = jnp.clip(pcx - 0.5 * pw, 0.0, img_w)
    y1 = jnp.clip(pcy - 0.5 * ph, 0.0, img_h)
    x2 = jnp.clip(pcx + 0.5 * pw, 0.0, img_w)
    y2 = jnp.clip(pcy + 0.5 * ph, 0.0, img_h)
    return x1, y1, x2, y2


def _make_body(img_w, img_h):
    def body(geo_c, geo_r, cls_c, cls_r, out_ref, xc, xr, rr, rc, sc, sr, keep):
        g = geo_c[...]
        bx1, by1, bx2, by2 = _boxes_from(
            g[:, 0:1], g[:, 1:2], g[:, 2:3], g[:, 3:4],
            g[:, 4:5], g[:, 5:6], g[:, 6:7], g[:, 7:8], img_w, img_h)
        c = cls_c[...]
        s_c = jnp.max(c, axis=1, keepdims=True)
        cl_c = jnp.min(
            jnp.where(c == s_c, _fiota((_NP, _CP), 1), 1e9),
            axis=1, keepdims=True)
        xc[:, 0:1] = bx1
        xc[:, 1:2] = by1
        xc[:, 2:3] = bx2
        xc[:, 3:4] = by2
        xc[:, 4:5] = s_c
        xc[:, 5:6] = cl_c
        xc[:, 6:8] = jnp.zeros((_NP, 2), jnp.float32)

        gr = geo_r[...]
        rx1, ry1, rx2, ry2 = _boxes_from(
            gr[0:1, :], gr[1:2, :], gr[2:3, :], gr[3:4, :],
            gr[4:5, :], gr[5:6, :], gr[6:7, :], gr[7:8, :], img_w, img_h)
        cr = cls_r[...]
        s_r = jnp.max(cr, axis=0, keepdims=True)
        cl_r = jnp.min(
            jnp.where(cr == s_r, _fiota((_CP, _NP), 0), 1e9),
            axis=0, keepdims=True)
        xr[0:1, :] = rx1
        xr[1:2, :] = ry1
        xr[2:3, :] = rx2
        xr[3:4, :] = ry2
        xr[4:5, :] = s_r
        xr[5:6, :] = cl_r
        xr[6:8, :] = jnp.zeros((2, _NP), jnp.float32)

        s_row = xr[4:5, :]
        s_col = xc[:, 4:5]
        for rb in range(_NB):
            s_i = xc[rb * _B:(rb + 1) * _B, 4:5]
            jg = _iota((_B, _NP), 1)
            ig = _iota((_B, _NP), 0) + rb * _B
            cmp = jnp.where(s_row > s_i, 1.0, 0.0) + jnp.where(
                (s_row == s_i) & (jg < ig), 1.0, 0.0)
            rc[rb * _B:(rb + 1) * _B, 0:1] = jnp.sum(cmp, axis=1, keepdims=True)
        for cb in range(_NB):
            s_i = xr[4:5, cb * _B:(cb + 1) * _B]
            jg = _iota((_NP, _B), 0)
            ig = _iota((_NP, _B), 1) + cb * _B
            cmp = jnp.where(s_col > s_i, 1.0, 0.0) + jnp.where(
                (s_col == s_i) & (jg < ig), 1.0, 0.0)
            rr[0:1, cb * _B:(cb + 1) * _B] = jnp.sum(cmp, axis=0, keepdims=True)

        rank_row = rr[0:1, :]
        rank_col = rc[:, 0:1]
        for rb in range(_NB):
            tgt = (_iota((_B, _NP), 0) + rb * _B).astype(jnp.float32)
            pblk = jnp.where(rank_row == tgt, 1.0, 0.0)
            sc[rb * _B:(rb + 1) * _B, :] = jnp.dot(
                pblk, xc[...], preferred_element_type=jnp.float32,
                precision=lax.Precision.HIGHEST)
        for cb in range(_NB):
            tgt = (_iota((_NP, _B), 1) + cb * _B).astype(jnp.float32)
            ptblk = jnp.where(rank_col == tgt, 1.0, 0.0)
            sr[:, cb * _B:(cb + 1) * _B] = jnp.dot(
                xr[...], ptblk, preferred_element_type=jnp.float32,
                precision=lax.Precision.HIGHEST)

        keep[:, :] = jnp.where(sr[4:5, :] > _ST, 1.0, 0.0)
        x1r = sr[0:1, :]
        y1r = sr[1:2, :]
        x2r = sr[2:3, :]
        y2r = sr[3:4, :]
        area_r = jnp.maximum(x2r - x1r, 0.0) * jnp.maximum(y2r - y1r, 0.0)
        for bi in range(_NB):
            lo = bi * _B
            hi = lo + _B
            x1b = sc[lo:hi, 0:1]
            y1b = sc[lo:hi, 1:2]
            x2b = sc[lo:hi, 2:3]
            y2b = sc[lo:hi, 3:4]
            area_b = jnp.maximum(x2b - x1b, 0.0) * jnp.maximum(y2b - y1b, 0.0)
            ix1 = jnp.maximum(x1b, x1r)
            iy1 = jnp.maximum(y1b, y1r)
            ix2 = jnp.minimum(x2b, x2r)
            iy2 = jnp.minimum(y2b, y2r)
            inter = jnp.maximum(ix2 - ix1, 0.0) * jnp.maximum(iy2 - iy1, 0.0)
            union = area_b + area_r - inter
            iou = inter / jnp.maximum(union, 1e-8)
            m = jnp.where(iou > _TH, 1.0, 0.0)
            mself = lax.slice(m, (0, lo), (_B, hi))
            tri = jnp.where(_iota((_B, _B), 1) > _iota((_B, _B), 0), 1.0, 0.0)
            mu = mself * tri
            kb0 = keep[0:1, lo:hi]

            def cond(carry):
                return carry[1]

            def step(carry):
                kb, _ = carry
                sup = jnp.dot(kb, mu, preferred_element_type=jnp.float32)
                kbn = kb0 * jnp.where(sup > 0.0, 0.0, 1.0)
                ch = jnp.sum(jnp.abs(kbn - kb)) > 0.0
                return (kbn, ch)

            kb_f, _ = lax.while_loop(cond, step, (kb0, jnp.bool_(True)))
            keep[0:1, lo:hi] = kb_f
            if bi < _NB - 1:
                sup_all = jnp.dot(kb_f, m, preferred_element_type=jnp.float32)
                later = _iota((1, _NP), 1) >= hi
                keep[:, :] = jnp.where(later & (sup_all > 0.0), 0.0, keep[...])

        k = keep[...]
        out_ref[0:4, :] = sr[0:4, :] * k
        out_ref[4:5, :] = sr[4:5, :] * k
        out_ref[5:6, :] = jnp.where(k > 0.0, sr[5:6, :], 0.0)
        out_ref[6:8, :] = jnp.zeros((2, _NP), jnp.float32)

    return body


def kernel(x, regression, classification, anchors):
    img_h = float(x.shape[2])
    img_w = float(x.shape[3])
    reg = regression[0]
    cls = classification[0]
    anc = anchors[0]
    geo = jnp.concatenate([anc, reg], axis=1)
    geo_c = jnp.pad(geo, ((0, _NP - _N), (0, 0)))
    cls_c = jnp.pad(cls, ((0, _NP - _N), (0, _CP - _NC)),
                    constant_values=-1.0)
    geo_r = geo_c.T
    cls_r = cls_c.T

    out = pl.pallas_call(
        _make_body(img_w, img_h),
        out_shape=jax.ShapeDtypeStruct((8, _NP), jnp.float32),
        scratch_shapes=[
            pltpu.VMEM((_NP, 8), jnp.float32),
            pltpu.VMEM((8, _NP), jnp.float32),
            pltpu.VMEM((1, _NP), jnp.float32),
            pltpu.VMEM((_NP, 1), jnp.float32),
            pltpu.VMEM((_NP, 8), jnp.float32),
            pltpu.VMEM((8, _NP), jnp.float32),
            pltpu.VMEM((1, _NP), jnp.float32),
        ],
    )(geo_c, geo_r, cls_c, cls_r)

    boxes = out[0:4, :_N].T
    scores = out[4, :_N]
    classes = out[5, :_N].astype(jnp.int32)
    return boxes, scores, classes
